# baseline (speedup 1.0000x reference)
import functools

import jax
import jax.numpy as jnp
from jax import lax
from jax.experimental import pallas as pl
from jax.experimental.pallas import tpu as pltpu
from jax.experimental.pallas import tpu_sc as plsc

B, S, D, E, K, F = 1, 2048, 768, 8, 2, 3072
T = B * S
BM = 256
EP = 128
NB = (T * K + E * (BM - 1)) // BM
NP = NB * BM
NBP = -(-NB // 8) * 8


def _router_body(x_ref, wr_ref, br_ref, pos1_ref, pos2_ref, w1_ref, w2_ref,
                 be_ref, vld_ref):
    x = x_ref[...]
    logits = jnp.dot(x, wr_ref[...], preferred_element_type=jnp.float32)
    logits = logits + br_ref[...]
    lane = lax.broadcasted_iota(jnp.int32, (T, EP), 1)
    neg = jnp.float32(-1e30)
    logits = jnp.where(lane < E, logits, neg)
    mx = jnp.max(logits, axis=1, keepdims=True)
    ex = jnp.where(lane < E, jnp.exp(logits - mx), 0.0)
    probs = ex / jnp.sum(ex, axis=1, keepdims=True)
    pm = jnp.where(lane < E, probs, neg)
    m1 = jnp.max(pm, axis=1, keepdims=True)
    i1 = jnp.min(jnp.where(pm == m1, lane, EP), axis=1, keepdims=True)
    pm2 = jnp.where(lane == i1, neg, pm)
    m2 = jnp.max(pm2, axis=1, keepdims=True)
    i2 = jnp.min(jnp.where(pm2 == m2, lane, EP), axis=1, keepdims=True)
    s = m1 + m2
    w1_ref[...] = m1 / s
    w2_ref[...] = m2 / s
    oh1 = lane == i1
    oh2 = lane == i2
    m = (oh1 | oh2).astype(jnp.float32)
    c = m
    sh = 1
    while sh < T:
        c = c + jnp.concatenate(
            [jnp.zeros((sh, EP), jnp.float32), c[:-sh, :]], axis=0)
        sh *= 2
    counts = c[T - 1:T, :]
    pc = jnp.ceil(counts * (1.0 / BM)) * BM
    ip = pc
    for d in (1, 2, 4):
        ip = ip + jnp.concatenate(
            [jnp.zeros((1, d), jnp.float32), ip[:, :-d]], axis=1)
    excl = ip - pc
    exb = jnp.broadcast_to(excl, (T, EP))
    pos1 = jnp.sum(jnp.where(oh1, exb + c - 1.0, 0.0), axis=1, keepdims=True)
    pos2 = jnp.sum(jnp.where(oh2, exb + c - 1.0, 0.0), axis=1, keepdims=True)
    pos1_ref[...] = pos1.astype(jnp.int32)
    pos2_ref[...] = pos2.astype(jnp.int32)
    ipb = jnp.broadcast_to(ip, (NBP, EP))
    brow = lax.broadcasted_iota(jnp.int32, (NBP, EP), 0).astype(jnp.float32) * BM
    lane32 = lax.broadcasted_iota(jnp.int32, (NBP, EP), 1)
    be = jnp.sum(jnp.where(lane32 < E, (brow >= ipb).astype(jnp.int32), 0),
                 axis=1, keepdims=True)
    tot = jnp.sum(jnp.where(lane == E - 1, jnp.broadcast_to(ip, (T, EP)), 0.0)[:1, :],
                  axis=1, keepdims=True)
    be_ref[...] = jnp.minimum(be, E - 1)
    vld_ref[...] = (brow[:, :1] < tot).astype(jnp.int32)


def _gmm_body(be_ref, vld_ref, xs_ref, w1_ref, b1_ref, w2_ref, b2_ref, ys_ref):
    i = pl.program_id(0)

    @pl.when(vld_ref[i] == 1)
    def _():
        xb = xs_ref[...].astype(jnp.bfloat16)
        h = jnp.dot(xb, w1_ref[0], preferred_element_type=jnp.float32)
        h = jax.nn.gelu(h + b1_ref[0], approximate=True)
        ys_ref[...] = (jnp.dot(h.astype(jnp.bfloat16), w2_ref[0],
                               preferred_element_type=jnp.float32)
                       + b2_ref[0])


def _mix_body(w1_ref, w2_ref, y1_ref, y2_ref, o_ref):
    o_ref[...] = w1_ref[...] * y1_ref[...] + w2_ref[...] * y2_ref[...]


def kernel(hidden_states, Wr, br, W1, b1, W2, b2):
    x = hidden_states.reshape(T, D)
    wrp = jnp.zeros((D, EP), jnp.float32).at[:, :E].set(Wr)
    brp = jnp.zeros((1, EP), jnp.float32).at[0, :E].set(br)

    pos1, pos2, w1, w2, be32, vld32 = pl.pallas_call(
        _router_body,
        out_shape=(
            jax.ShapeDtypeStruct((T, 1), jnp.int32),
            jax.ShapeDtypeStruct((T, 1), jnp.int32),
            jax.ShapeDtypeStruct((T, 1), jnp.float32),
            jax.ShapeDtypeStruct((T, 1), jnp.float32),
            jax.ShapeDtypeStruct((NBP, 1), jnp.int32),
            jax.ShapeDtypeStruct((NBP, 1), jnp.int32),
        ),
    )(x, wrp, brp)

    pos1 = pos1.reshape(T)
    pos2 = pos2.reshape(T)
    be = be32[:NB, 0]
    vld = vld32[:NB, 0]

    mesh = plsc.VectorSubcoreMesh(core_axis_name="c", subcore_axis_name="s")
    nw = mesh.num_cores * mesh.num_subcores
    tw = T // nw

    @functools.partial(
        pl.kernel,
        mesh=mesh,
        out_type=jax.ShapeDtypeStruct((NP, D), jnp.float32),
        scratch_types=[
            pltpu.VMEM((tw,), jnp.int32),
            pltpu.VMEM((tw,), jnp.int32),
            pltpu.VMEM((tw, D), jnp.float32),
            pltpu.SemaphoreType.DMA,
            pltpu.SemaphoreType.DMA,
        ],
    )
    def _dispatch(x_hbm, p1_hbm, p2_hbm, xs_hbm, i1_v, i2_v, rows_v, sem_a, sem_b):
        wid = lax.axis_index("s") * mesh.num_cores + lax.axis_index("c")
        base = wid * tw
        c_rows = pltpu.async_copy(x_hbm.at[pl.ds(base, tw)], rows_v, sem_a)
        c_i1 = pltpu.async_copy(p1_hbm.at[pl.ds(base, tw)], i1_v, sem_b)
        c_i2 = pltpu.async_copy(p2_hbm.at[pl.ds(base, tw)], i2_v, sem_b)
        c_rows.wait()
        c_i1.wait()
        c_i2.wait()
        s1 = pltpu.async_copy(rows_v, xs_hbm.at[i1_v], sem_a)
        s2 = pltpu.async_copy(rows_v, xs_hbm.at[i2_v], sem_b)
        s1.wait()
        s2.wait()

    xs = _dispatch(x, pos1, pos2)

    grid_spec = pltpu.PrefetchScalarGridSpec(
        num_scalar_prefetch=2,
        grid=(NB,),
        in_specs=[
            pl.BlockSpec((BM, D), lambda i, be_r, v_r: (i, 0)),
            pl.BlockSpec((1, D, F), lambda i, be_r, v_r: (be_r[i], 0, 0)),
            pl.BlockSpec((1, 1, F), lambda i, be_r, v_r: (be_r[i], 0, 0)),
            pl.BlockSpec((1, F, D), lambda i, be_r, v_r: (be_r[i], 0, 0)),
            pl.BlockSpec((1, 1, D), lambda i, be_r, v_r: (be_r[i], 0, 0)),
        ],
        out_specs=pl.BlockSpec((BM, D), lambda i, be_r, v_r: (i, 0)),
    )
    ys = pl.pallas_call(
        _gmm_body,
        grid_spec=grid_spec,
        out_shape=jax.ShapeDtypeStruct((NP, D), jnp.float32),
        compiler_params=pltpu.CompilerParams(
            dimension_semantics=("arbitrary",)),
    )(be, vld, xs, W1.astype(jnp.bfloat16), b1.reshape(E, 1, F),
      W2.astype(jnp.bfloat16), b2.reshape(E, 1, D))

    @functools.partial(
        pl.kernel,
        mesh=mesh,
        out_type=(
            jax.ShapeDtypeStruct((T, D), jnp.float32),
            jax.ShapeDtypeStruct((T, D), jnp.float32),
        ),
        scratch_types=[
            pltpu.VMEM((tw,), jnp.int32),
            pltpu.VMEM((tw,), jnp.int32),
            pltpu.VMEM((tw, D), jnp.float32),
            pltpu.VMEM((tw, D), jnp.float32),
            pltpu.SemaphoreType.DMA,
            pltpu.SemaphoreType.DMA,
        ],
    )
    def _combine(ys_hbm, p1_hbm, p2_hbm, y1_hbm, y2_hbm, i1_v, i2_v, g1_v,
                 g2_v, sem_a, sem_b):
        wid = lax.axis_index("s") * mesh.num_cores + lax.axis_index("c")
        base = wid * tw
        c_i1 = pltpu.async_copy(p1_hbm.at[pl.ds(base, tw)], i1_v, sem_a)
        c_i2 = pltpu.async_copy(p2_hbm.at[pl.ds(base, tw)], i2_v, sem_b)
        c_i1.wait()
        c_i2.wait()
        g1 = pltpu.async_copy(ys_hbm.at[i1_v], g1_v, sem_a)
        g2 = pltpu.async_copy(ys_hbm.at[i2_v], g2_v, sem_b)
        g1.wait()
        g2.wait()
        o1 = pltpu.async_copy(g1_v, y1_hbm.at[pl.ds(base, tw)], sem_a)
        o2 = pltpu.async_copy(g2_v, y2_hbm.at[pl.ds(base, tw)], sem_b)
        o1.wait()
        o2.wait()

    y1, y2 = _combine(ys, pos1, pos2)

    out = pl.pallas_call(
        _mix_body,
        out_shape=jax.ShapeDtypeStruct((T, D), jnp.float32),
    )(w1, w2, y1, y2)
    return out.reshape(B, S, D)


# speedup vs baseline: 1.2692x; 1.2692x over previous
import functools

import jax
import jax.numpy as jnp
from jax import lax
from jax.experimental import pallas as pl
from jax.experimental.pallas import tpu as pltpu
from jax.experimental.pallas import tpu_sc as plsc

B, S, D, E, K, F = 1, 2048, 768, 8, 2, 3072
T = B * S
BM = 256
EP = 128
NB = (T * K + E * (BM - 1)) // BM
NP = NB * BM
NBP = -(-NB // 8) * 8


def _router_body(x_ref, wr_ref, br_ref, pos1_ref, pos2_ref, w1_ref, w2_ref,
                 be_ref, vld_ref):
    x = x_ref[...]
    logits = jnp.dot(x, wr_ref[...], preferred_element_type=jnp.float32)
    logits = logits + br_ref[...]
    lane = lax.broadcasted_iota(jnp.int32, (T, EP), 1)
    neg = jnp.float32(-1e30)
    logits = jnp.where(lane < E, logits, neg)
    mx = jnp.max(logits, axis=1, keepdims=True)
    ex = jnp.where(lane < E, jnp.exp(logits - mx), 0.0)
    probs = ex / jnp.sum(ex, axis=1, keepdims=True)
    pm = jnp.where(lane < E, probs, neg)
    m1 = jnp.max(pm, axis=1, keepdims=True)
    i1 = jnp.min(jnp.where(pm == m1, lane, EP), axis=1, keepdims=True)
    pm2 = jnp.where(lane == i1, neg, pm)
    m2 = jnp.max(pm2, axis=1, keepdims=True)
    i2 = jnp.min(jnp.where(pm2 == m2, lane, EP), axis=1, keepdims=True)
    s = m1 + m2
    w1_ref[...] = m1 / s
    w2_ref[...] = m2 / s
    oh1 = lane == i1
    oh2 = lane == i2
    m = (oh1 | oh2).astype(jnp.float32)
    c = m
    sh = 1
    while sh < T:
        c = c + jnp.concatenate(
            [jnp.zeros((sh, EP), jnp.float32), c[:-sh, :]], axis=0)
        sh *= 2
    counts = c[T - 1:T, :]
    pc = jnp.ceil(counts * (1.0 / BM)) * BM
    ip = pc
    for d in (1, 2, 4):
        ip = ip + jnp.concatenate(
            [jnp.zeros((1, d), jnp.float32), ip[:, :-d]], axis=1)
    excl = ip - pc
    exb = jnp.broadcast_to(excl, (T, EP))
    pos1 = jnp.sum(jnp.where(oh1, exb + c - 1.0, 0.0), axis=1, keepdims=True)
    pos2 = jnp.sum(jnp.where(oh2, exb + c - 1.0, 0.0), axis=1, keepdims=True)
    pos1_ref[...] = pos1.astype(jnp.int32)
    pos2_ref[...] = pos2.astype(jnp.int32)
    ipb = jnp.broadcast_to(ip, (NBP, EP))
    brow = lax.broadcasted_iota(jnp.int32, (NBP, EP), 0).astype(jnp.float32) * BM
    lane32 = lax.broadcasted_iota(jnp.int32, (NBP, EP), 1)
    be = jnp.sum(jnp.where(lane32 < E, (brow >= ipb).astype(jnp.int32), 0),
                 axis=1, keepdims=True)
    tot = jnp.sum(jnp.where(lane == E - 1, jnp.broadcast_to(ip, (T, EP)), 0.0)[:1, :],
                  axis=1, keepdims=True)
    be_ref[...] = jnp.minimum(be, E - 1)
    vld_ref[...] = (brow[:, :1] < tot).astype(jnp.int32)


def _gmm_body(be_ref, vld_ref, xs_ref, w1_ref, b1_ref, w2_ref, b2_ref, ys_ref):
    i = pl.program_id(0)

    @pl.when(vld_ref[i] == 1)
    def _():
        xb = xs_ref[...].astype(jnp.bfloat16)
        h = jnp.dot(xb, w1_ref[0].astype(jnp.bfloat16),
                    preferred_element_type=jnp.float32)
        h = jax.nn.gelu(h + b1_ref[0], approximate=True)
        ys_ref[...] = (jnp.dot(h.astype(jnp.bfloat16),
                               w2_ref[0].astype(jnp.bfloat16),
                               preferred_element_type=jnp.float32)
                       + b2_ref[0])


def _mix_body(w1_ref, w2_ref, y1_ref, y2_ref, o_ref):
    o_ref[...] = w1_ref[...] * y1_ref[...] + w2_ref[...] * y2_ref[...]


def kernel(hidden_states, Wr, br, W1, b1, W2, b2):
    x = hidden_states.reshape(T, D)
    wrp = jnp.zeros((D, EP), jnp.float32).at[:, :E].set(Wr)
    brp = jnp.zeros((1, EP), jnp.float32).at[0, :E].set(br)

    pos1, pos2, w1, w2, be32, vld32 = pl.pallas_call(
        _router_body,
        out_shape=(
            jax.ShapeDtypeStruct((T, 1), jnp.int32),
            jax.ShapeDtypeStruct((T, 1), jnp.int32),
            jax.ShapeDtypeStruct((T, 1), jnp.float32),
            jax.ShapeDtypeStruct((T, 1), jnp.float32),
            jax.ShapeDtypeStruct((NBP, 1), jnp.int32),
            jax.ShapeDtypeStruct((NBP, 1), jnp.int32),
        ),
    )(x, wrp, brp)

    pos1 = pos1.reshape(T)
    pos2 = pos2.reshape(T)
    be = be32[:NB, 0]
    vld = vld32[:NB, 0]

    mesh = plsc.VectorSubcoreMesh(core_axis_name="c", subcore_axis_name="s")
    nw = mesh.num_cores * mesh.num_subcores
    tw = T // nw

    @functools.partial(
        pl.kernel,
        mesh=mesh,
        out_type=jax.ShapeDtypeStruct((NP, D), jnp.float32),
        scratch_types=[
            pltpu.VMEM((tw,), jnp.int32),
            pltpu.VMEM((tw,), jnp.int32),
            pltpu.VMEM((tw, D), jnp.float32),
            pltpu.SemaphoreType.DMA,
            pltpu.SemaphoreType.DMA,
        ],
    )
    def _dispatch(x_hbm, p1_hbm, p2_hbm, xs_hbm, i1_v, i2_v, rows_v, sem_a, sem_b):
        wid = lax.axis_index("s") * mesh.num_cores + lax.axis_index("c")
        base = wid * tw
        c_rows = pltpu.async_copy(x_hbm.at[pl.ds(base, tw)], rows_v, sem_a)
        c_i1 = pltpu.async_copy(p1_hbm.at[pl.ds(base, tw)], i1_v, sem_b)
        c_i2 = pltpu.async_copy(p2_hbm.at[pl.ds(base, tw)], i2_v, sem_b)
        c_rows.wait()
        c_i1.wait()
        c_i2.wait()
        s1 = pltpu.async_copy(rows_v, xs_hbm.at[i1_v], sem_a)
        s2 = pltpu.async_copy(rows_v, xs_hbm.at[i2_v], sem_b)
        s1.wait()
        s2.wait()

    xs = _dispatch(x, pos1, pos2)

    grid_spec = pltpu.PrefetchScalarGridSpec(
        num_scalar_prefetch=2,
        grid=(NB,),
        in_specs=[
            pl.BlockSpec((BM, D), lambda i, be_r, v_r: (i, 0)),
            pl.BlockSpec((1, D, F), lambda i, be_r, v_r: (be_r[i], 0, 0)),
            pl.BlockSpec((1, 1, F), lambda i, be_r, v_r: (be_r[i], 0, 0)),
            pl.BlockSpec((1, F, D), lambda i, be_r, v_r: (be_r[i], 0, 0)),
            pl.BlockSpec((1, 1, D), lambda i, be_r, v_r: (be_r[i], 0, 0)),
        ],
        out_specs=pl.BlockSpec((BM, D), lambda i, be_r, v_r: (i, 0)),
    )
    ys = pl.pallas_call(
        _gmm_body,
        grid_spec=grid_spec,
        out_shape=jax.ShapeDtypeStruct((NP, D), jnp.float32),
        compiler_params=pltpu.CompilerParams(
            dimension_semantics=("arbitrary",)),
    )(be, vld, xs, W1, b1.reshape(E, 1, F), W2, b2.reshape(E, 1, D))

    @functools.partial(
        pl.kernel,
        mesh=mesh,
        out_type=(
            jax.ShapeDtypeStruct((T, D), jnp.float32),
            jax.ShapeDtypeStruct((T, D), jnp.float32),
        ),
        scratch_types=[
            pltpu.VMEM((tw,), jnp.int32),
            pltpu.VMEM((tw,), jnp.int32),
            pltpu.VMEM((tw, D), jnp.float32),
            pltpu.VMEM((tw, D), jnp.float32),
            pltpu.SemaphoreType.DMA,
            pltpu.SemaphoreType.DMA,
        ],
    )
    def _combine(ys_hbm, p1_hbm, p2_hbm, y1_hbm, y2_hbm, i1_v, i2_v, g1_v,
                 g2_v, sem_a, sem_b):
        wid = lax.axis_index("s") * mesh.num_cores + lax.axis_index("c")
        base = wid * tw
        c_i1 = pltpu.async_copy(p1_hbm.at[pl.ds(base, tw)], i1_v, sem_a)
        c_i2 = pltpu.async_copy(p2_hbm.at[pl.ds(base, tw)], i2_v, sem_b)
        c_i1.wait()
        c_i2.wait()
        g1 = pltpu.async_copy(ys_hbm.at[i1_v], g1_v, sem_a)
        g2 = pltpu.async_copy(ys_hbm.at[i2_v], g2_v, sem_b)
        g1.wait()
        g2.wait()
        o1 = pltpu.async_copy(g1_v, y1_hbm.at[pl.ds(base, tw)], sem_a)
        o2 = pltpu.async_copy(g2_v, y2_hbm.at[pl.ds(base, tw)], sem_b)
        o1.wait()
        o2.wait()

    y1, y2 = _combine(ys, pos1, pos2)

    out = pl.pallas_call(
        _mix_body,
        out_shape=jax.ShapeDtypeStruct((T, D), jnp.float32),
    )(w1, w2, y1, y2)
    return out.reshape(B, S, D)


# speedup vs baseline: 1.2896x; 1.0161x over previous
import functools

import jax
import jax.numpy as jnp
from jax import lax
from jax.experimental import pallas as pl
from jax.experimental.pallas import tpu as pltpu
from jax.experimental.pallas import tpu_sc as plsc

B, S, D, E, K, F = 1, 2048, 768, 8, 2, 3072
T = B * S
BM = 256
EP = 128
NB = (T * K + E * (BM - 1)) // BM
NP = NB * BM
NBP = -(-NB // 8) * 8


def _router_body(x_ref, wr_ref, br_ref, pos1_ref, pos2_ref, w1_ref, w2_ref,
                 be_ref, vld_ref):
    x = x_ref[...]
    logits = jnp.dot(x, wr_ref[...], preferred_element_type=jnp.float32)
    logits = logits + br_ref[...]
    lane = lax.broadcasted_iota(jnp.int32, (T, EP), 1)
    neg = jnp.float32(-1e30)
    logits = jnp.where(lane < E, logits, neg)
    mx = jnp.max(logits, axis=1, keepdims=True)
    ex = jnp.where(lane < E, jnp.exp(logits - mx), 0.0)
    probs = ex / jnp.sum(ex, axis=1, keepdims=True)
    pm = jnp.where(lane < E, probs, neg)
    m1 = jnp.max(pm, axis=1, keepdims=True)
    i1 = jnp.min(jnp.where(pm == m1, lane, EP), axis=1, keepdims=True)
    pm2 = jnp.where(lane == i1, neg, pm)
    m2 = jnp.max(pm2, axis=1, keepdims=True)
    i2 = jnp.min(jnp.where(pm2 == m2, lane, EP), axis=1, keepdims=True)
    s = m1 + m2
    w1_ref[...] = m1 / s
    w2_ref[...] = m2 / s
    oh1 = lane == i1
    oh2 = lane == i2
    m = (oh1 | oh2).astype(jnp.float32)
    c = m
    sh = 1
    while sh < T:
        c = c + jnp.concatenate(
            [jnp.zeros((sh, EP), jnp.float32), c[:-sh, :]], axis=0)
        sh *= 2
    counts = c[T - 1:T, :]
    pc = jnp.ceil(counts * (1.0 / BM)) * BM
    ip = pc
    for d in (1, 2, 4):
        ip = ip + jnp.concatenate(
            [jnp.zeros((1, d), jnp.float32), ip[:, :-d]], axis=1)
    excl = ip - pc
    exb = jnp.broadcast_to(excl, (T, EP))
    pos1 = jnp.sum(jnp.where(oh1, exb + c - 1.0, 0.0), axis=1, keepdims=True)
    pos2 = jnp.sum(jnp.where(oh2, exb + c - 1.0, 0.0), axis=1, keepdims=True)
    pos1_ref[...] = pos1.astype(jnp.int32)
    pos2_ref[...] = pos2.astype(jnp.int32)
    ipb = jnp.broadcast_to(ip, (NBP, EP))
    brow = lax.broadcasted_iota(jnp.int32, (NBP, EP), 0).astype(jnp.float32) * BM
    lane32 = lax.broadcasted_iota(jnp.int32, (NBP, EP), 1)
    be = jnp.sum(jnp.where(lane32 < E, (brow >= ipb).astype(jnp.int32), 0),
                 axis=1, keepdims=True)
    tot = jnp.sum(jnp.where(lane == E - 1, jnp.broadcast_to(ip, (T, EP)), 0.0)[:1, :],
                  axis=1, keepdims=True)
    be_ref[...] = jnp.minimum(be, E - 1)
    vld_ref[...] = (brow[:, :1] < tot).astype(jnp.int32)


def _gmm_body(be_ref, vld_ref, xs_ref, w1_ref, b1_ref, w2_ref, b2_ref, ys_ref):
    i = pl.program_id(0)

    @pl.when(vld_ref[i] == 1)
    def _():
        h = jnp.dot(xs_ref[...], w1_ref[0], preferred_element_type=jnp.float32)
        h = jax.nn.gelu(h + b1_ref[0], approximate=True)
        ys_ref[...] = (jnp.dot(h, w2_ref[0], preferred_element_type=jnp.float32)
                       + b2_ref[0])


def _mix_body(w1_ref, w2_ref, y1_ref, y2_ref, o_ref):
    o_ref[...] = w1_ref[...] * y1_ref[...] + w2_ref[...] * y2_ref[...]


def kernel(hidden_states, Wr, br, W1, b1, W2, b2):
    x = hidden_states.reshape(T, D)
    wrp = jnp.zeros((D, EP), jnp.float32).at[:, :E].set(Wr)
    brp = jnp.zeros((1, EP), jnp.float32).at[0, :E].set(br)

    pos1, pos2, w1, w2, be32, vld32 = pl.pallas_call(
        _router_body,
        out_shape=(
            jax.ShapeDtypeStruct((T, 1), jnp.int32),
            jax.ShapeDtypeStruct((T, 1), jnp.int32),
            jax.ShapeDtypeStruct((T, 1), jnp.float32),
            jax.ShapeDtypeStruct((T, 1), jnp.float32),
            jax.ShapeDtypeStruct((NBP, 1), jnp.int32),
            jax.ShapeDtypeStruct((NBP, 1), jnp.int32),
        ),
    )(x, wrp, brp)

    pos1 = pos1.reshape(T)
    pos2 = pos2.reshape(T)
    be = be32[:NB, 0]
    vld = vld32[:NB, 0]

    mesh = plsc.VectorSubcoreMesh(core_axis_name="c", subcore_axis_name="s")
    nw = mesh.num_cores * mesh.num_subcores
    tw = T // nw

    @functools.partial(
        pl.kernel,
        mesh=mesh,
        out_type=jax.ShapeDtypeStruct((NP, D), jnp.float32),
        scratch_types=[
            pltpu.VMEM((tw,), jnp.int32),
            pltpu.VMEM((tw,), jnp.int32),
            pltpu.VMEM((tw, D), jnp.float32),
            pltpu.SemaphoreType.DMA,
            pltpu.SemaphoreType.DMA,
        ],
    )
    def _dispatch(x_hbm, p1_hbm, p2_hbm, xs_hbm, i1_v, i2_v, rows_v, sem_a, sem_b):
        wid = lax.axis_index("s") * mesh.num_cores + lax.axis_index("c")
        base = wid * tw
        c_rows = pltpu.async_copy(x_hbm.at[pl.ds(base, tw)], rows_v, sem_a)
        c_i1 = pltpu.async_copy(p1_hbm.at[pl.ds(base, tw)], i1_v, sem_b)
        c_i2 = pltpu.async_copy(p2_hbm.at[pl.ds(base, tw)], i2_v, sem_b)
        c_rows.wait()
        c_i1.wait()
        c_i2.wait()
        s1 = pltpu.async_copy(rows_v, xs_hbm.at[i1_v], sem_a)
        s2 = pltpu.async_copy(rows_v, xs_hbm.at[i2_v], sem_b)
        s1.wait()
        s2.wait()

    xs = _dispatch(x, pos1, pos2)

    grid_spec = pltpu.PrefetchScalarGridSpec(
        num_scalar_prefetch=2,
        grid=(NB,),
        in_specs=[
            pl.BlockSpec((BM, D), lambda i, be_r, v_r: (i, 0)),
            pl.BlockSpec((1, D, F), lambda i, be_r, v_r: (be_r[i], 0, 0)),
            pl.BlockSpec((1, 1, F), lambda i, be_r, v_r: (be_r[i], 0, 0)),
            pl.BlockSpec((1, F, D), lambda i, be_r, v_r: (be_r[i], 0, 0)),
            pl.BlockSpec((1, 1, D), lambda i, be_r, v_r: (be_r[i], 0, 0)),
        ],
        out_specs=pl.BlockSpec((BM, D), lambda i, be_r, v_r: (i, 0)),
    )
    ys = pl.pallas_call(
        _gmm_body,
        grid_spec=grid_spec,
        out_shape=jax.ShapeDtypeStruct((NP, D), jnp.float32),
        compiler_params=pltpu.CompilerParams(
            dimension_semantics=("arbitrary",)),
    )(be, vld, xs, W1, b1.reshape(E, 1, F), W2, b2.reshape(E, 1, D))

    @functools.partial(
        pl.kernel,
        mesh=mesh,
        out_type=(
            jax.ShapeDtypeStruct((T, D), jnp.float32),
            jax.ShapeDtypeStruct((T, D), jnp.float32),
        ),
        scratch_types=[
            pltpu.VMEM((tw,), jnp.int32),
            pltpu.VMEM((tw,), jnp.int32),
            pltpu.VMEM((tw, D), jnp.float32),
            pltpu.VMEM((tw, D), jnp.float32),
            pltpu.SemaphoreType.DMA,
            pltpu.SemaphoreType.DMA,
        ],
    )
    def _combine(ys_hbm, p1_hbm, p2_hbm, y1_hbm, y2_hbm, i1_v, i2_v, g1_v,
                 g2_v, sem_a, sem_b):
        wid = lax.axis_index("s") * mesh.num_cores + lax.axis_index("c")
        base = wid * tw
        c_i1 = pltpu.async_copy(p1_hbm.at[pl.ds(base, tw)], i1_v, sem_a)
        c_i2 = pltpu.async_copy(p2_hbm.at[pl.ds(base, tw)], i2_v, sem_b)
        c_i1.wait()
        c_i2.wait()
        g1 = pltpu.async_copy(ys_hbm.at[i1_v], g1_v, sem_a)
        g2 = pltpu.async_copy(ys_hbm.at[i2_v], g2_v, sem_b)
        g1.wait()
        g2.wait()
        o1 = pltpu.async_copy(g1_v, y1_hbm.at[pl.ds(base, tw)], sem_a)
        o2 = pltpu.async_copy(g2_v, y2_hbm.at[pl.ds(base, tw)], sem_b)
        o1.wait()
        o2.wait()

    y1, y2 = _combine(ys, pos1, pos2)

    out = pl.pallas_call(
        _mix_body,
        out_shape=jax.ShapeDtypeStruct((T, D), jnp.float32),
    )(w1, w2, y1, y2)
    return out.reshape(B, S, D)


# speedup vs baseline: 1.3118x; 1.0172x over previous
import functools

import jax
import jax.numpy as jnp
from jax import lax
from jax.experimental import pallas as pl
from jax.experimental.pallas import tpu as pltpu
from jax.experimental.pallas import tpu_sc as plsc

B, S, D, E, K, F = 1, 2048, 768, 8, 2, 3072
T = B * S
BM = 256
EP = 128
NB = (T * K + E * (BM - 1)) // BM
NP = NB * BM
NBP = -(-NB // 8) * 8


def _router_body(x_ref, wr_ref, br_ref, pos1_ref, pos2_ref, w1_ref, w2_ref,
                 be_ref, vld_ref):
    x = x_ref[...]
    logits = jnp.dot(x, wr_ref[...], preferred_element_type=jnp.float32)
    logits = logits + br_ref[...]
    lane = lax.broadcasted_iota(jnp.int32, (T, EP), 1)
    neg = jnp.float32(-1e30)
    logits = jnp.where(lane < E, logits, neg)
    mx = jnp.max(logits, axis=1, keepdims=True)
    ex = jnp.where(lane < E, jnp.exp(logits - mx), 0.0)
    probs = ex / jnp.sum(ex, axis=1, keepdims=True)
    pm = jnp.where(lane < E, probs, neg)
    m1 = jnp.max(pm, axis=1, keepdims=True)
    i1 = jnp.min(jnp.where(pm == m1, lane, EP), axis=1, keepdims=True)
    pm2 = jnp.where(lane == i1, neg, pm)
    m2 = jnp.max(pm2, axis=1, keepdims=True)
    i2 = jnp.min(jnp.where(pm2 == m2, lane, EP), axis=1, keepdims=True)
    s = m1 + m2
    w1_ref[...] = m1 / s
    w2_ref[...] = m2 / s
    oh1 = lane == i1
    oh2 = lane == i2
    m = (oh1 | oh2).astype(jnp.float32)
    c = m
    sh = 1
    while sh < T:
        c = c + jnp.concatenate(
            [jnp.zeros((sh, EP), jnp.float32), c[:-sh, :]], axis=0)
        sh *= 2
    counts = c[T - 1:T, :]
    pc = jnp.ceil(counts * (1.0 / BM)) * BM
    ip = pc
    for d in (1, 2, 4):
        ip = ip + jnp.concatenate(
            [jnp.zeros((1, d), jnp.float32), ip[:, :-d]], axis=1)
    excl = ip - pc
    exb = jnp.broadcast_to(excl, (T, EP))
    pos1 = jnp.sum(jnp.where(oh1, exb + c - 1.0, 0.0), axis=1, keepdims=True)
    pos2 = jnp.sum(jnp.where(oh2, exb + c - 1.0, 0.0), axis=1, keepdims=True)
    pos1_ref[...] = pos1.astype(jnp.int32)
    pos2_ref[...] = pos2.astype(jnp.int32)
    ipb = jnp.broadcast_to(ip, (NBP, EP))
    brow = lax.broadcasted_iota(jnp.int32, (NBP, EP), 0).astype(jnp.float32) * BM
    lane32 = lax.broadcasted_iota(jnp.int32, (NBP, EP), 1)
    be = jnp.sum(jnp.where(lane32 < E, (brow >= ipb).astype(jnp.int32), 0),
                 axis=1, keepdims=True)
    tot = jnp.sum(jnp.where(lane == E - 1, jnp.broadcast_to(ip, (T, EP)), 0.0)[:1, :],
                  axis=1, keepdims=True)
    be_ref[...] = jnp.minimum(be, E - 1)
    vld_ref[...] = (brow[:, :1] < tot).astype(jnp.int32)


def _gmm_body(be_ref, vld_ref, xs_ref, w1_ref, b1_ref, w2_ref, b2_ref, ys_ref):
    i = pl.program_id(0)

    @pl.when(vld_ref[i] == 1)
    def _():
        h = jnp.dot(xs_ref[...], w1_ref[0], preferred_element_type=jnp.float32)
        h = jax.nn.gelu(h + b1_ref[0], approximate=True)
        ys_ref[...] = (jnp.dot(h, w2_ref[0], preferred_element_type=jnp.float32)
                       + b2_ref[0])


def kernel(hidden_states, Wr, br, W1, b1, W2, b2):
    x = hidden_states.reshape(T, D)
    wrp = jnp.zeros((D, EP), jnp.float32).at[:, :E].set(Wr)
    brp = jnp.zeros((1, EP), jnp.float32).at[0, :E].set(br)

    pos1, pos2, w1, w2, be32, vld32 = pl.pallas_call(
        _router_body,
        out_shape=(
            jax.ShapeDtypeStruct((T, 1), jnp.int32),
            jax.ShapeDtypeStruct((T, 1), jnp.int32),
            jax.ShapeDtypeStruct((T, 1), jnp.float32),
            jax.ShapeDtypeStruct((T, 1), jnp.float32),
            jax.ShapeDtypeStruct((NBP, 1), jnp.int32),
            jax.ShapeDtypeStruct((NBP, 1), jnp.int32),
        ),
    )(x, wrp, brp)

    pos1 = pos1.reshape(T)
    pos2 = pos2.reshape(T)
    be = be32.reshape(NBP)
    vld = vld32.reshape(NBP)

    mesh = plsc.VectorSubcoreMesh(core_axis_name="c", subcore_axis_name="s")
    nw = mesh.num_cores * mesh.num_subcores
    tw = T // nw

    @functools.partial(
        pl.kernel,
        mesh=mesh,
        out_type=jax.ShapeDtypeStruct((NP, D), jnp.float32),
        scratch_types=[
            pltpu.VMEM((tw,), jnp.int32),
            pltpu.VMEM((tw,), jnp.int32),
            pltpu.VMEM((tw, D), jnp.float32),
            pltpu.SemaphoreType.DMA,
            pltpu.SemaphoreType.DMA,
        ],
    )
    def _dispatch(x_hbm, p1_hbm, p2_hbm, xs_hbm, i1_v, i2_v, rows_v, sem_a, sem_b):
        wid = lax.axis_index("s") * mesh.num_cores + lax.axis_index("c")
        base = wid * tw
        c_rows = pltpu.async_copy(x_hbm.at[pl.ds(base, tw)], rows_v, sem_a)
        c_i1 = pltpu.async_copy(p1_hbm.at[pl.ds(base, tw)], i1_v, sem_b)
        c_i2 = pltpu.async_copy(p2_hbm.at[pl.ds(base, tw)], i2_v, sem_b)
        c_rows.wait()
        c_i1.wait()
        c_i2.wait()
        s1 = pltpu.async_copy(rows_v, xs_hbm.at[i1_v], sem_a)
        s2 = pltpu.async_copy(rows_v, xs_hbm.at[i2_v], sem_b)
        s1.wait()
        s2.wait()

    xs = _dispatch(x, pos1, pos2)

    grid_spec = pltpu.PrefetchScalarGridSpec(
        num_scalar_prefetch=2,
        grid=(NB,),
        in_specs=[
            pl.BlockSpec((BM, D), lambda i, be_r, v_r: (i, 0)),
            pl.BlockSpec((1, D, F), lambda i, be_r, v_r: (be_r[i], 0, 0)),
            pl.BlockSpec((1, 1, F), lambda i, be_r, v_r: (be_r[i], 0, 0)),
            pl.BlockSpec((1, F, D), lambda i, be_r, v_r: (be_r[i], 0, 0)),
            pl.BlockSpec((1, 1, D), lambda i, be_r, v_r: (be_r[i], 0, 0)),
        ],
        out_specs=pl.BlockSpec((BM, D), lambda i, be_r, v_r: (i, 0)),
    )
    ys = pl.pallas_call(
        _gmm_body,
        grid_spec=grid_spec,
        out_shape=jax.ShapeDtypeStruct((NP, D), jnp.float32),
        compiler_params=pltpu.CompilerParams(
            dimension_semantics=("arbitrary",)),
    )(be, vld, xs, W1, b1.reshape(E, 1, F), W2, b2.reshape(E, 1, D))

    @functools.partial(
        pl.kernel,
        mesh=mesh,
        out_type=jax.ShapeDtypeStruct((T, D), jnp.float32),
        scratch_types=[
            pltpu.VMEM((tw,), jnp.int32),
            pltpu.VMEM((tw,), jnp.int32),
            pltpu.VMEM((tw,), jnp.float32),
            pltpu.VMEM((tw,), jnp.float32),
            pltpu.VMEM((tw, D), jnp.float32),
            pltpu.VMEM((tw, D), jnp.float32),
            pltpu.SemaphoreType.DMA,
            pltpu.SemaphoreType.DMA,
        ],
    )
    def _combine(ys_hbm, p1_hbm, p2_hbm, w1_hbm, w2_hbm, o_hbm, i1_v, i2_v,
                 w1_v, w2_v, g1_v, g2_v, sem_a, sem_b):
        wid = lax.axis_index("s") * mesh.num_cores + lax.axis_index("c")
        base = wid * tw
        c_i1 = pltpu.async_copy(p1_hbm.at[pl.ds(base, tw)], i1_v, sem_a)
        c_i2 = pltpu.async_copy(p2_hbm.at[pl.ds(base, tw)], i2_v, sem_b)
        c_w1 = pltpu.async_copy(w1_hbm.at[pl.ds(base, tw)], w1_v, sem_a)
        c_w2 = pltpu.async_copy(w2_hbm.at[pl.ds(base, tw)], w2_v, sem_b)
        c_i1.wait()
        c_i2.wait()
        g1 = pltpu.async_copy(ys_hbm.at[i1_v], g1_v, sem_a)
        g2 = pltpu.async_copy(ys_hbm.at[i2_v], g2_v, sem_b)
        c_w1.wait()
        c_w2.wait()
        g1.wait()
        g2.wait()

        def group_body(g, _):
            wa = w1_v[pl.ds(g * 16, 16)]
            wb = w2_v[pl.ds(g * 16, 16)]
            for r in range(16):
                i = g * 16 + r
                a = wa[r]
                b_ = wb[r]

                def chunk_body(j, _):
                    sl = pl.ds(j * 64, 16)
                    sl2 = pl.ds(j * 64 + 16, 16)
                    sl3 = pl.ds(j * 64 + 32, 16)
                    sl4 = pl.ds(j * 64 + 48, 16)
                    g1_v[i, sl] = g1_v[i, sl] * a + g2_v[i, sl] * b_
                    g1_v[i, sl2] = g1_v[i, sl2] * a + g2_v[i, sl2] * b_
                    g1_v[i, sl3] = g1_v[i, sl3] * a + g2_v[i, sl3] * b_
                    g1_v[i, sl4] = g1_v[i, sl4] * a + g2_v[i, sl4] * b_
                    return 0

                lax.fori_loop(0, D // 64, chunk_body, 0)
            return 0

        lax.fori_loop(0, tw // 16, group_body, 0)
        pltpu.sync_copy(g1_v, o_hbm.at[pl.ds(base, tw)])

    out = _combine(ys, pos1, pos2, w1.reshape(T), w2.reshape(T))
    return out.reshape(B, S, D)


# speedup vs baseline: 1.3410x; 1.0223x over previous
import functools

import jax
import jax.numpy as jnp
from jax import lax
from jax.experimental import pallas as pl
from jax.experimental.pallas import tpu as pltpu
from jax.experimental.pallas import tpu_sc as plsc

B, S, D, E, K, F = 1, 2048, 768, 8, 2, 3072
T = B * S
BM = 256
EP = E
NB = (T * K + E * (BM - 1)) // BM
NP = NB * BM
NBP = -(-NB // 8) * 8


def _router_body(x_ref, wr_ref, br_ref, pos1_ref, pos2_ref, w1_ref, w2_ref,
                 be_ref, vld_ref):
    x = x_ref[...]
    logits = jnp.dot(x, wr_ref[...], preferred_element_type=jnp.float32)
    logits = logits + br_ref[...]
    lane = lax.broadcasted_iota(jnp.int32, (T, EP), 1)
    neg = jnp.float32(-1e30)
    mx = jnp.max(logits, axis=1, keepdims=True)
    ex = jnp.exp(logits - mx)
    probs = ex / jnp.sum(ex, axis=1, keepdims=True)
    pm = jnp.where(lane < E, probs, neg)
    m1 = jnp.max(pm, axis=1, keepdims=True)
    i1 = jnp.min(jnp.where(pm == m1, lane, EP), axis=1, keepdims=True)
    pm2 = jnp.where(lane == i1, neg, pm)
    m2 = jnp.max(pm2, axis=1, keepdims=True)
    i2 = jnp.min(jnp.where(pm2 == m2, lane, EP), axis=1, keepdims=True)
    s = m1 + m2
    w1_ref[...] = m1 / s
    w2_ref[...] = m2 / s
    oh1 = lane == i1
    oh2 = lane == i2
    m = (oh1 | oh2).astype(jnp.float32)
    c = m
    sh = 1
    while sh < T:
        c = c + jnp.concatenate(
            [jnp.zeros((sh, EP), jnp.float32), c[:-sh, :]], axis=0)
        sh *= 2
    counts = c[T - 1:T, :]
    pc = jnp.ceil(counts * (1.0 / BM)) * BM
    ip = pc
    for d in (1, 2, 4):
        ip = ip + jnp.concatenate(
            [jnp.zeros((1, d), jnp.float32), ip[:, :-d]], axis=1)
    excl = ip - pc
    exb = jnp.broadcast_to(excl, (T, EP))
    pos1 = jnp.sum(jnp.where(oh1, exb + c - 1.0, 0.0), axis=1, keepdims=True)
    pos2 = jnp.sum(jnp.where(oh2, exb + c - 1.0, 0.0), axis=1, keepdims=True)
    pos1_ref[...] = pos1.astype(jnp.int32)
    pos2_ref[...] = pos2.astype(jnp.int32)
    ipb = jnp.broadcast_to(ip, (NBP, EP))
    brow = lax.broadcasted_iota(jnp.int32, (NBP, EP), 0).astype(jnp.float32) * BM
    lane32 = lax.broadcasted_iota(jnp.int32, (NBP, EP), 1)
    be = jnp.sum(jnp.where(lane32 < E, (brow >= ipb).astype(jnp.int32), 0),
                 axis=1, keepdims=True)
    tot = jnp.sum(jnp.where(lane == E - 1, jnp.broadcast_to(ip, (T, EP)), 0.0)[:1, :],
                  axis=1, keepdims=True)
    be_ref[...] = jnp.minimum(be, E - 1)
    vld_ref[...] = (brow[:, :1] < tot).astype(jnp.int32)


def _gmm_body(be_ref, vld_ref, xs_ref, w1_ref, b1_ref, w2_ref, b2_ref, ys_ref):
    i = pl.program_id(0)

    @pl.when(vld_ref[i] == 1)
    def _():
        h = jnp.dot(xs_ref[...], w1_ref[0], preferred_element_type=jnp.float32)
        h = jax.nn.gelu(h + b1_ref[0], approximate=True)
        ys_ref[...] = (jnp.dot(h, w2_ref[0], preferred_element_type=jnp.float32)
                       + b2_ref[0])


def kernel(hidden_states, Wr, br, W1, b1, W2, b2):
    x = hidden_states.reshape(T, D)

    pos1, pos2, w1, w2, be32, vld32 = pl.pallas_call(
        _router_body,
        out_shape=(
            jax.ShapeDtypeStruct((T, 1), jnp.int32),
            jax.ShapeDtypeStruct((T, 1), jnp.int32),
            jax.ShapeDtypeStruct((T, 1), jnp.float32),
            jax.ShapeDtypeStruct((T, 1), jnp.float32),
            jax.ShapeDtypeStruct((NBP, 1), jnp.int32),
            jax.ShapeDtypeStruct((NBP, 1), jnp.int32),
        ),
    )(x, Wr, br.reshape(1, E))

    pos1 = pos1.reshape(T)
    pos2 = pos2.reshape(T)
    be = be32.reshape(NBP)
    vld = vld32.reshape(NBP)

    mesh = plsc.VectorSubcoreMesh(core_axis_name="c", subcore_axis_name="s")
    nw = mesh.num_cores * mesh.num_subcores
    tw = T // nw

    @functools.partial(
        pl.kernel,
        mesh=mesh,
        out_type=jax.ShapeDtypeStruct((NP, D), jnp.float32),
        scratch_types=[
            pltpu.VMEM((tw,), jnp.int32),
            pltpu.VMEM((tw,), jnp.int32),
            pltpu.VMEM((tw, D), jnp.float32),
            pltpu.SemaphoreType.DMA,
            pltpu.SemaphoreType.DMA,
        ],
    )
    def _dispatch(x_hbm, p1_hbm, p2_hbm, xs_hbm, i1_v, i2_v, rows_v, sem_a, sem_b):
        wid = lax.axis_index("s") * mesh.num_cores + lax.axis_index("c")
        base = wid * tw
        c_rows = pltpu.async_copy(x_hbm.at[pl.ds(base, tw)], rows_v, sem_a)
        c_i1 = pltpu.async_copy(p1_hbm.at[pl.ds(base, tw)], i1_v, sem_b)
        c_i2 = pltpu.async_copy(p2_hbm.at[pl.ds(base, tw)], i2_v, sem_b)
        c_rows.wait()
        c_i1.wait()
        c_i2.wait()
        s1 = pltpu.async_copy(rows_v, xs_hbm.at[i1_v], sem_a)
        s2 = pltpu.async_copy(rows_v, xs_hbm.at[i2_v], sem_b)
        s1.wait()
        s2.wait()

    xs = _dispatch(x, pos1, pos2)

    grid_spec = pltpu.PrefetchScalarGridSpec(
        num_scalar_prefetch=2,
        grid=(NB,),
        in_specs=[
            pl.BlockSpec((BM, D), lambda i, be_r, v_r: (i, 0)),
            pl.BlockSpec((1, D, F), lambda i, be_r, v_r: (be_r[i], 0, 0)),
            pl.BlockSpec((1, 1, F), lambda i, be_r, v_r: (be_r[i], 0, 0)),
            pl.BlockSpec((1, F, D), lambda i, be_r, v_r: (be_r[i], 0, 0)),
            pl.BlockSpec((1, 1, D), lambda i, be_r, v_r: (be_r[i], 0, 0)),
        ],
        out_specs=pl.BlockSpec((BM, D), lambda i, be_r, v_r: (i, 0)),
    )
    ys = pl.pallas_call(
        _gmm_body,
        grid_spec=grid_spec,
        out_shape=jax.ShapeDtypeStruct((NP, D), jnp.float32),
        compiler_params=pltpu.CompilerParams(
            dimension_semantics=("arbitrary",)),
    )(be, vld, xs, W1, b1.reshape(E, 1, F), W2, b2.reshape(E, 1, D))

    @functools.partial(
        pl.kernel,
        mesh=mesh,
        out_type=jax.ShapeDtypeStruct((T, D), jnp.float32),
        scratch_types=[
            pltpu.VMEM((tw,), jnp.int32),
            pltpu.VMEM((tw,), jnp.int32),
            pltpu.VMEM((tw,), jnp.float32),
            pltpu.VMEM((tw,), jnp.float32),
            pltpu.VMEM((tw, D), jnp.float32),
            pltpu.VMEM((tw, D), jnp.float32),
            pltpu.SemaphoreType.DMA,
            pltpu.SemaphoreType.DMA,
        ],
    )
    def _combine(ys_hbm, p1_hbm, p2_hbm, w1_hbm, w2_hbm, o_hbm, i1_v, i2_v,
                 w1_v, w2_v, g1_v, g2_v, sem_a, sem_b):
        wid = lax.axis_index("s") * mesh.num_cores + lax.axis_index("c")
        base = wid * tw
        c_i1 = pltpu.async_copy(p1_hbm.at[pl.ds(base, tw)], i1_v, sem_a)
        c_i2 = pltpu.async_copy(p2_hbm.at[pl.ds(base, tw)], i2_v, sem_b)
        c_w1 = pltpu.async_copy(w1_hbm.at[pl.ds(base, tw)], w1_v, sem_a)
        c_w2 = pltpu.async_copy(w2_hbm.at[pl.ds(base, tw)], w2_v, sem_b)
        c_i1.wait()
        c_i2.wait()
        g1 = pltpu.async_copy(ys_hbm.at[i1_v], g1_v, sem_a)
        g2 = pltpu.async_copy(ys_hbm.at[i2_v], g2_v, sem_b)
        c_w1.wait()
        c_w2.wait()
        g1.wait()
        g2.wait()

        def group_body(g, _):
            wa = w1_v[pl.ds(g * 16, 16)]
            wb = w2_v[pl.ds(g * 16, 16)]
            for r in range(16):
                i = g * 16 + r
                a = wa[r]
                b_ = wb[r]

                def chunk_body(j, _):
                    sl = pl.ds(j * 64, 16)
                    sl2 = pl.ds(j * 64 + 16, 16)
                    sl3 = pl.ds(j * 64 + 32, 16)
                    sl4 = pl.ds(j * 64 + 48, 16)
                    g1_v[i, sl] = g1_v[i, sl] * a + g2_v[i, sl] * b_
                    g1_v[i, sl2] = g1_v[i, sl2] * a + g2_v[i, sl2] * b_
                    g1_v[i, sl3] = g1_v[i, sl3] * a + g2_v[i, sl3] * b_
                    g1_v[i, sl4] = g1_v[i, sl4] * a + g2_v[i, sl4] * b_
                    return 0

                lax.fori_loop(0, D // 64, chunk_body, 0)
            return 0

        lax.fori_loop(0, tw // 16, group_body, 0)
        pltpu.sync_copy(g1_v, o_hbm.at[pl.ds(base, tw)])

    out = _combine(ys, pos1, pos2, w1.reshape(T), w2.reshape(T))
    return out.reshape(B, S, D)
